# baseline (speedup 1.0000x reference)
import functools

import jax
import jax.numpy as jnp
from jax import lax
from jax.experimental import pallas as pl
from jax.experimental.pallas import tpu as pltpu
from jax.experimental.pallas import tpu_sc as plsc

K = 8
B = 32
H = 32768
C = 64
L = 16
NG = C // L
CHH = 512
NCHUNK = H // CHH
RB = 32
NBLK = CHH // RB


def _neg_inf():
    return jnp.full((L,), -jnp.inf, dtype=jnp.float32)


def _kmax_body(x_hbm, out_hbm, buf0, buf1, state, sem0, sem1):
    cid = lax.axis_index("c")
    sid = lax.axis_index("s")
    b = sid * 2 + cid

    for i in range(K):
        for g in range(NG):
            state[i, pl.ds(g * L, L)] = _neg_inf()

    iota = lax.iota(jnp.int32, L)
    idx_c = [iota + jnp.full((L,), g * L, jnp.int32) for g in range(NG)]

    bufs = (buf0, buf1)
    sems = (sem0, sem1)

    def start_dma(chunk, slot):
        pltpu.make_async_copy(
            x_hbm.at[b, :, pl.ds(chunk * CHH, CHH)], bufs[slot], sems[slot]
        ).start()

    def wait_dma(chunk, slot):
        pltpu.make_async_copy(
            x_hbm.at[b, :, pl.ds(chunk * CHH, CHH)], bufs[slot], sems[slot]
        ).wait()

    start_dma(0, 0)
    start_dma(1, 1)

    def process_chunk(slot, r7s):
        bufr = bufs[slot]

        def load(g, idx_h):
            return plsc.load_gather(bufr, [idx_c[g], idx_h])

        def skew_idx(base, kk):
            skew = (iota + jnp.full((L,), kk, jnp.int32)) & jnp.full(
                (L,), RB - 1, jnp.int32
            )
            return skew + jnp.full((L,), base, jnp.int32)

        def block_masks(base, r7s_in):
            idx0 = skew_idx(base, 0)
            idx1 = skew_idx(base, 1)
            macc_a = [load(g, idx0) for g in range(NG)]
            macc_b = [load(g, idx1) for g in range(NG)]
            for kk in range(2, RB, 2):
                idx0 = skew_idx(base, kk)
                idx1 = skew_idx(base, kk + 1)
                for g in range(NG):
                    macc_a[g] = jnp.maximum(macc_a[g], load(g, idx0))
                    macc_b[g] = jnp.maximum(macc_b[g], load(g, idx1))
            return [
                jnp.maximum(macc_a[g], macc_b[g]) > r7s_in[g]
                for g in range(NG)
            ]

        def insert_block(base, g):
            s = [state[i, pl.ds(g * L, L)] for i in range(K)]
            for kk in range(RB):
                t = load(g, skew_idx(base, kk))
                for i in range(K):
                    hi = jnp.maximum(s[i], t)
                    t = jnp.minimum(s[i], t)
                    s[i] = hi
            for i in range(K):
                state[i, pl.ds(g * L, L)] = s[i]
            return s[K - 1]

        def block(j, r7s_in):
            base = j * RB
            ms = block_masks(base, r7s_in)
            hit = jnp.any((ms[0] | ms[1]) | (ms[2] | ms[3]))

            def slow(rs):
                anys = [jnp.any(m) for m in ms]
                outs = []
                for g in range(NG):
                    def ins(g=g):
                        return insert_block(base, g)

                    outs.append(lax.cond(anys[g], ins, lambda g=g: rs[g]))
                return tuple(outs)

            return lax.cond(hit, slow, lambda rs: rs, r7s_in)

        return lax.fori_loop(0, NBLK, block, r7s)

    def outer(i, r7s):
        c0 = 2 * i
        wait_dma(c0, 0)
        r7s = process_chunk(0, r7s)

        @pl.when(c0 + 2 < NCHUNK)
        def _():
            start_dma(c0 + 2, 0)

        wait_dma(c0 + 1, 1)
        r7s = process_chunk(1, r7s)

        @pl.when(c0 + 3 < NCHUNK)
        def _():
            start_dma(c0 + 3, 1)

        return r7s

    r7s = tuple(_neg_inf() for _ in range(NG))
    lax.fori_loop(0, NCHUNK // 2, outer, r7s)

    pltpu.sync_copy(state, out_hbm.at[b])


@jax.jit
def _kmax(x):
    mesh = plsc.VectorSubcoreMesh(core_axis_name="c", subcore_axis_name="s")
    f = functools.partial(
        pl.kernel,
        out_type=jax.ShapeDtypeStruct((B, K, C), jnp.float32),
        mesh=mesh,
        compiler_params=pltpu.CompilerParams(needs_layout_passes=False),
        scratch_types=[
            pltpu.VMEM((C, CHH), jnp.float32),
            pltpu.VMEM((C, CHH), jnp.float32),
            pltpu.VMEM((K, C), jnp.float32),
            pltpu.SemaphoreType.DMA,
            pltpu.SemaphoreType.DMA,
        ],
    )(_kmax_body)
    return f(x)


def kernel(inputs):
    x = jnp.transpose(inputs, (0, 2, 3, 1)).reshape(B, C, H)
    out = _kmax(x)
    return out.reshape(B, K, 1, C)


# speedup vs baseline: 1.2959x; 1.2959x over previous
import functools

import jax
import jax.numpy as jnp
from jax import lax
from jax.experimental import pallas as pl
from jax.experimental.pallas import tpu as pltpu
from jax.experimental.pallas import tpu_sc as plsc

K = 8
B = 32
H = 32768
C = 64
L = 16
NG = C // L
CHH = 512
NCHUNK = H // CHH
RB = 16
NBLK = CHH // RB


def _neg_inf():
    return jnp.full((L,), -jnp.inf, dtype=jnp.float32)


def _kmax_body(x_hbm, out_hbm, buf0, buf1, state, sem0, sem1):
    cid = lax.axis_index("c")
    sid = lax.axis_index("s")
    b = sid * 2 + cid

    for i in range(K):
        for g in range(NG):
            state[i, pl.ds(g * L, L)] = _neg_inf()

    iota = lax.iota(jnp.int32, L)
    idx_c = [iota + jnp.full((L,), g * L, jnp.int32) for g in range(NG)]

    bufs = (buf0, buf1)
    sems = (sem0, sem1)

    def start_dma(chunk, slot):
        pltpu.make_async_copy(
            x_hbm.at[b, :, pl.ds(chunk * CHH, CHH)], bufs[slot], sems[slot]
        ).start()

    def wait_dma(chunk, slot):
        pltpu.make_async_copy(
            x_hbm.at[b, :, pl.ds(chunk * CHH, CHH)], bufs[slot], sems[slot]
        ).wait()

    start_dma(0, 0)
    start_dma(1, 1)

    def process_chunk(slot, r7s):
        bufr = bufs[slot]

        def load(g, idx_h):
            return plsc.load_gather(bufr, [idx_c[g], idx_h])

        def skew_idx(base, kk):
            skew = (iota + jnp.full((L,), kk, jnp.int32)) & jnp.full(
                (L,), RB - 1, jnp.int32
            )
            return skew + jnp.full((L,), base, jnp.int32)

        def block_masks(base, r7s_in):
            idxs = [skew_idx(base, kk) for kk in range(4)]
            accs = [[load(g, idxs[a]) for g in range(NG)] for a in range(4)]
            for kk in range(4, RB, 4):
                idxs = [skew_idx(base, kk + a) for a in range(4)]
                for a in range(4):
                    for g in range(NG):
                        accs[a][g] = jnp.maximum(
                            accs[a][g], load(g, idxs[a])
                        )
            return [
                jnp.maximum(
                    jnp.maximum(accs[0][g], accs[1][g]),
                    jnp.maximum(accs[2][g], accs[3][g]),
                )
                > r7s_in[g]
                for g in range(NG)
            ]

        def insert_block(base, g):
            s = [state[i, pl.ds(g * L, L)] for i in range(K)]
            for kk in range(RB):
                t = load(g, skew_idx(base, kk))
                for i in range(K):
                    hi = jnp.maximum(s[i], t)
                    t = jnp.minimum(s[i], t)
                    s[i] = hi
            for i in range(K):
                state[i, pl.ds(g * L, L)] = s[i]
            return s[K - 1]

        def block(j, r7s_in):
            base = j * RB
            ms = block_masks(base, r7s_in)
            hit = jnp.any((ms[0] | ms[1]) | (ms[2] | ms[3]))

            def slow(rs):
                anys = [jnp.any(m) for m in ms]
                outs = []
                for g in range(NG):
                    def ins(g=g):
                        return insert_block(base, g)

                    outs.append(lax.cond(anys[g], ins, lambda g=g: rs[g]))
                return tuple(outs)

            return lax.cond(hit, slow, lambda rs: rs, r7s_in)

        return lax.fori_loop(0, NBLK, block, r7s)

    def outer(i, r7s):
        c0 = 2 * i
        wait_dma(c0, 0)
        r7s = process_chunk(0, r7s)

        @pl.when(c0 + 2 < NCHUNK)
        def _():
            start_dma(c0 + 2, 0)

        wait_dma(c0 + 1, 1)
        r7s = process_chunk(1, r7s)

        @pl.when(c0 + 3 < NCHUNK)
        def _():
            start_dma(c0 + 3, 1)

        return r7s

    r7s = tuple(_neg_inf() for _ in range(NG))
    lax.fori_loop(0, NCHUNK // 2, outer, r7s)

    pltpu.sync_copy(state, out_hbm.at[b])


@jax.jit
def _kmax(x):
    mesh = plsc.VectorSubcoreMesh(core_axis_name="c", subcore_axis_name="s")
    f = functools.partial(
        pl.kernel,
        out_type=jax.ShapeDtypeStruct((B, K, C), jnp.float32),
        mesh=mesh,
        compiler_params=pltpu.CompilerParams(needs_layout_passes=False),
        scratch_types=[
            pltpu.VMEM((C, CHH), jnp.float32),
            pltpu.VMEM((C, CHH), jnp.float32),
            pltpu.VMEM((K, C), jnp.float32),
            pltpu.SemaphoreType.DMA,
            pltpu.SemaphoreType.DMA,
        ],
    )(_kmax_body)
    return f(x)


def kernel(inputs):
    x = jnp.transpose(inputs, (0, 2, 3, 1)).reshape(B, C, H)
    out = _kmax(x)
    return out.reshape(B, K, 1, C)


# speedup vs baseline: 1.3813x; 1.0659x over previous
import functools

import jax
import jax.numpy as jnp
from jax import lax
from jax.experimental import pallas as pl
from jax.experimental.pallas import tpu as pltpu
from jax.experimental.pallas import tpu_sc as plsc

K = 8
B = 32
H = 32768
C = 64
L = 16
NG = C // L
CHH = 512
NCHUNK = H // CHH
RB = 16
NBLK = CHH // RB


def _neg_inf():
    return jnp.full((L,), -jnp.inf, dtype=jnp.float32)


def _kmax_body(x_hbm, out_hbm, buf0, buf1, state, sem0, sem1):
    cid = lax.axis_index("c")
    sid = lax.axis_index("s")
    b = sid * 2 + cid

    for i in range(K):
        for g in range(NG):
            state[i, pl.ds(g * L, L)] = _neg_inf()

    iota = lax.iota(jnp.int32, L)
    idx_c = [iota + jnp.full((L,), g * L, jnp.int32) for g in range(NG)]

    bufs = (buf0, buf1)
    sems = (sem0, sem1)

    def start_dma(chunk, slot):
        pltpu.make_async_copy(
            x_hbm.at[b, :, pl.ds(chunk * CHH, CHH)], bufs[slot], sems[slot]
        ).start()

    def wait_dma(chunk, slot):
        pltpu.make_async_copy(
            x_hbm.at[b, :, pl.ds(chunk * CHH, CHH)], bufs[slot], sems[slot]
        ).wait()

    start_dma(0, 0)
    start_dma(1, 1)

    def process_chunk(slot, r7s):
        bufr = bufs[slot]

        def load(g, idx_h):
            return plsc.load_gather(bufr, [idx_c[g], idx_h])

        def skew_idx(base, kk):
            skew = (iota + jnp.full((L,), kk, jnp.int32)) & jnp.full(
                (L,), RB - 1, jnp.int32
            )
            return skew + jnp.full((L,), base, jnp.int32)

        def block_masks(base, r7s_in):
            idx0 = skew_idx(base, 0)
            idx1 = skew_idx(base, 1)
            macc_a = [load(g, idx0) for g in range(NG)]
            macc_b = [load(g, idx1) for g in range(NG)]
            for kk in range(2, RB, 2):
                idx0 = skew_idx(base, kk)
                idx1 = skew_idx(base, kk + 1)
                for g in range(NG):
                    macc_a[g] = jnp.maximum(macc_a[g], load(g, idx0))
                    macc_b[g] = jnp.maximum(macc_b[g], load(g, idx1))
            return [
                jnp.maximum(macc_a[g], macc_b[g]) > r7s_in[g]
                for g in range(NG)
            ]

        def ce(a, bb):
            return jnp.maximum(a, bb), jnp.minimum(a, bb)

        def bitonic8_desc(m):
            for stride in (4, 2, 1):
                for i in range(8):
                    if (i & stride) == 0 and i + stride < 8:
                        hi_v, lo_v = ce(m[i], m[i + stride])
                        m[i], m[i + stride] = hi_v, lo_v
            return m

        def insert_block(base, g):
            v = [load(g, skew_idx(base, kk)) for kk in range(RB)]
            p = []
            for i in range(8):
                hi, lo = ce(v[2 * i], v[2 * i + 1])
                p.append([hi, lo])
            q = []
            for i in range(4):
                x, y = p[2 * i], p[2 * i + 1]
                z0, t1 = ce(x[0], y[0])
                t2, z3 = ce(x[1], y[1])
                z1, z2 = ce(t1, t2)
                q.append([z0, z1, z2, z3])
            r = []
            for i in range(2):
                x, y = q[2 * i], q[2 * i + 1]
                r.append(
                    bitonic8_desc(
                        [x[0], x[1], x[2], x[3], y[3], y[2], y[1], y[0]]
                    )
                )
            m = [jnp.maximum(r[0][i], r[1][7 - i]) for i in range(8)]
            m = bitonic8_desc(m)
            s = [state[i, pl.ds(g * L, L)] for i in range(K)]
            m2 = [jnp.maximum(s[i], m[7 - i]) for i in range(8)]
            m2 = bitonic8_desc(m2)
            for i in range(K):
                state[i, pl.ds(g * L, L)] = m2[i]
            return m2[K - 1]

        def block(j, r7s_in):
            base = j * RB
            ms = block_masks(base, r7s_in)
            hit = jnp.any((ms[0] | ms[1]) | (ms[2] | ms[3]))

            def slow(rs):
                anys = [jnp.any(m) for m in ms]
                outs = []
                for g in range(NG):
                    def ins(g=g):
                        return insert_block(base, g)

                    outs.append(lax.cond(anys[g], ins, lambda g=g: rs[g]))
                return tuple(outs)

            return lax.cond(hit, slow, lambda rs: rs, r7s_in)

        return lax.fori_loop(0, NBLK, block, r7s)

    def outer(i, r7s):
        c0 = 2 * i
        wait_dma(c0, 0)
        r7s = process_chunk(0, r7s)

        @pl.when(c0 + 2 < NCHUNK)
        def _():
            start_dma(c0 + 2, 0)

        wait_dma(c0 + 1, 1)
        r7s = process_chunk(1, r7s)

        @pl.when(c0 + 3 < NCHUNK)
        def _():
            start_dma(c0 + 3, 1)

        return r7s

    r7s = tuple(_neg_inf() for _ in range(NG))
    lax.fori_loop(0, NCHUNK // 2, outer, r7s)

    pltpu.sync_copy(state, out_hbm.at[b])


@jax.jit
def _kmax(x):
    mesh = plsc.VectorSubcoreMesh(core_axis_name="c", subcore_axis_name="s")
    f = functools.partial(
        pl.kernel,
        out_type=jax.ShapeDtypeStruct((B, K, C), jnp.float32),
        mesh=mesh,
        compiler_params=pltpu.CompilerParams(needs_layout_passes=False),
        scratch_types=[
            pltpu.VMEM((C, CHH), jnp.float32),
            pltpu.VMEM((C, CHH), jnp.float32),
            pltpu.VMEM((K, C), jnp.float32),
            pltpu.SemaphoreType.DMA,
            pltpu.SemaphoreType.DMA,
        ],
    )(_kmax_body)
    return f(x)


def kernel(inputs):
    x = jnp.transpose(inputs, (0, 2, 3, 1)).reshape(B, C, H)
    out = _kmax(x)
    return out.reshape(B, K, 1, C)
